# baseline (speedup 1.0000x reference)
import functools

import jax
import jax.numpy as jnp
from jax import lax
from jax.experimental import pallas as pl
from jax.experimental.pallas import tpu as pltpu
from jax.experimental.pallas import tpu_sc as plsc

NC = 2
NS = 16
NW = NC * NS


def _make_gather(N, E, H):
    EPW = E // NW
    GCH = 1000
    NCH = EPW // GCH
    assert EPW % GCH == 0 and EPW % 8 == 0 and GCH % 8 == 0

    mesh = plsc.VectorSubcoreMesh(core_axis_name="c", subcore_axis_name="s")

    @functools.partial(
        pl.kernel, mesh=mesh,
        out_type=jax.ShapeDtypeStruct((E, H), jnp.float32),
        scratch_types=[
            pltpu.VMEM((EPW,), jnp.int32),
            pltpu.VMEM((2, GCH, H), jnp.float32),
            pltpu.SemaphoreType.DMA,
            pltpu.SemaphoreType.DMA,
            pltpu.SemaphoreType.DMA,
            pltpu.SemaphoreType.DMA,
        ],
        compiler_params=pltpu.CompilerParams(use_tc_tiling_on_sc=False),
    )
    def gather_k(h_hbm, esrc_hbm, out_hbm, idx_v, rows_v, g0, g1, w0, w1):
        c = lax.axis_index("c")
        s = lax.axis_index("s")
        w = c * NS + s
        base = w * EPW
        gsem = (g0, g1)
        wsem = (w0, w1)
        pltpu.sync_copy(esrc_hbm.at[pl.ds(base, EPW)], idx_v)
        gd = [None, None]
        wd = [None, None]
        prev = None
        for k in range(NCH):
            b = k & 1
            if k >= 2:
                wd[b].wait()
            gd[b] = pltpu.async_copy(
                h_hbm.at[idx_v.at[pl.ds(k * GCH, GCH)]], rows_v.at[b], gsem[b])
            if prev is not None:
                pk, pb = prev
                gd[pb].wait()
                wd[pb] = pltpu.async_copy(
                    rows_v.at[pb], out_hbm.at[pl.ds(base + pk * GCH, GCH)],
                    wsem[pb])
            prev = (k, b)
        pk, pb = prev
        gd[pb].wait()
        wd[pb] = pltpu.async_copy(
            rows_v.at[pb], out_hbm.at[pl.ds(base + pk * GCH, GCH)], wsem[pb])
        for b in range(2):
            if wd[b] is not None:
                wd[b].wait()

    return gather_k


def _make_scatter(N, E, H):
    SB = 40
    RPW = (E // SB) // NW
    MCH = 1000
    IRC = MCH // SB
    NCH = (RPW * SB) // MCH
    NPT = N // NS
    assert RPW * SB * NW == E and NCH * MCH == RPW * SB and N % NS == 0

    mesh = plsc.VectorSubcoreMesh(core_axis_name="c", subcore_axis_name="s")

    @functools.partial(
        pl.kernel, mesh=mesh,
        out_type=jax.ShapeDtypeStruct((NC, N, H), jnp.float32),
        scratch_types=[
            pltpu.VMEM((2, MCH, H), jnp.float32),
            pltpu.VMEM((2, IRC, SB), jnp.int32),
            pltpu.VMEM_SHARED((N, H), jnp.float32),
            pltpu.SemaphoreType.DMA,
            pltpu.SemaphoreType.DMA,
            pltpu.SemaphoreType.DMA,
            pltpu.SemaphoreType.DMA,
            pltpu.SemaphoreType.DMA,
        ],
        compiler_params=pltpu.CompilerParams(use_tc_tiling_on_sc=False),
    )
    def scatter_k(msg_hbm, etgt2_hbm, zeros_hbm, out_hbm,
                  mbuf, ibuf, acc_sh, m0, m1, i0, i1, ssem):
        c = lax.axis_index("c")
        s = lax.axis_index("s")
        w = c * NS + s
        ebase = w * RPW * SB
        rbase = w * RPW
        msem = (m0, m1)
        isem = (i0, i1)
        pltpu.sync_copy(zeros_hbm.at[pl.ds(s * NPT, NPT)],
                        acc_sh.at[pl.ds(s * NPT, NPT)])
        plsc.subcore_barrier()

        def start_stage(k, b):
            d1 = pltpu.async_copy(
                msg_hbm.at[pl.ds(ebase + k * MCH, MCH)], mbuf.at[b], msem[b])
            d2 = pltpu.async_copy(
                etgt2_hbm.at[pl.ds(rbase + k * IRC, IRC)], ibuf.at[b], isem[b])
            return (d1, d2)

        def fire_scatters(b):
            descs = []
            for j in range(IRC):
                descs.append(pltpu.async_copy(
                    mbuf.at[b].at[pl.ds(j * SB, SB)],
                    acc_sh.at[ibuf.at[b].at[j]],
                    ssem, add=True))
            return descs

        scat = {}
        prev = None
        for k in range(NCH):
            b = k & 1
            if k >= 2:
                for d in scat.pop(k - 2):
                    d.wait()
            sd = start_stage(k, b)
            if prev is not None:
                pk, pb, psd = prev
                psd[0].wait()
                psd[1].wait()
                scat[pk] = fire_scatters(pb)
            prev = (k, b, sd)
        pk, pb, psd = prev
        psd[0].wait()
        psd[1].wait()
        scat[pk] = fire_scatters(pb)
        for k in sorted(scat):
            for d in scat[k]:
                d.wait()
        plsc.subcore_barrier()
        pltpu.sync_copy(acc_sh.at[pl.ds(s * NPT, NPT)],
                        out_hbm.at[c].at[pl.ds(s * NPT, NPT)])

    return scatter_k


def _bmm_sub(ed_halves, sblk, r2, H, PB):
    Q = PB // 4
    sT = jnp.swapaxes(sblk, 0, 1)
    supt = jnp.concatenate(
        [sT[q * H:(q + 1) * H, :] for q in range(4)],
        axis=1).astype(jnp.bfloat16)
    mt = None
    c0 = 0
    for ed in ed_halves:
        rows = ed.shape[0] * H
        prod = (ed * supt[None]).reshape(rows, PB)
        part = jnp.dot(r2[:, c0:c0 + rows], prod,
                       preferred_element_type=jnp.float32)
        mt = part if mt is None else mt + part
        c0 += rows
    outs = [jnp.swapaxes(mt[:, q * Q:(q + 1) * Q], 0, 1)
            for q in range(4)]
    return jnp.concatenate(outs, axis=1)


def _make_bmm(E, H, BE, PB):
    HH = H * H
    NSUB = BE // PB

    def bmm_body(eda_ref, edb_ref, sup4_ref, r2_ref, out_ref):
        for t in range(NSUB):
            eda = eda_ref[:, t * PB:(t + 1) * PB].reshape(H // 2, H, PB)
            edb = edb_ref[:, t * PB:(t + 1) * PB].reshape(H // 2, H, PB)
            sblk = sup4_ref[t * (PB // 4):(t + 1) * (PB // 4), :]
            out_ref[t * (PB // 4):(t + 1) * (PB // 4), :] = _bmm_sub(
                [eda, edb], sblk, r2_ref[...], H, PB)

    grid = (E // BE,)
    return pl.pallas_call(
        bmm_body,
        grid=grid,
        in_specs=[
            pl.BlockSpec((HH // 2, BE), lambda k: (0, k)),
            pl.BlockSpec((HH // 2, BE), lambda k: (1, k)),
            pl.BlockSpec((BE // 4, 128), lambda k: (k, 0)),
            pl.BlockSpec((H, HH), lambda k: (0, 0)),
        ],
        out_specs=pl.BlockSpec((BE // 4, 128), lambda k: (k, 0)),
        out_shape=jax.ShapeDtypeStruct((E // 4, 128), jnp.float32),
        compiler_params=pltpu.CompilerParams(
            vmem_limit_bytes=56 * 1024 * 1024),
    )


def _make_bmm_first(E, H, BE, PB):
    HH = H * H
    NSUB = BE // PB

    def bmm_body(edt_ref, sup4_ref, r2_ref, out_ref, edtbf_ref):
        for t in range(NSUB):
            ed3 = edt_ref[:, t * PB:(t + 1) * PB].reshape(H, H, PB)
            ed_bf = ed3.astype(jnp.bfloat16)
            edtbf_ref[:, t * PB:(t + 1) * PB] = ed_bf.reshape(HH, PB)
            sblk = sup4_ref[t * (PB // 4):(t + 1) * (PB // 4), :]
            out_ref[t * (PB // 4):(t + 1) * (PB // 4), :] = _bmm_sub(
                [ed_bf], sblk, r2_ref[...], H, PB)

    grid = (E // BE,)
    return pl.pallas_call(
        bmm_body,
        grid=grid,
        in_specs=[
            pl.BlockSpec((HH, BE), lambda k: (0, k)),
            pl.BlockSpec((BE // 4, 128), lambda k: (k, 0)),
            pl.BlockSpec((H, HH), lambda k: (0, 0)),
        ],
        out_specs=[
            pl.BlockSpec((BE // 4, 128), lambda k: (k, 0)),
            pl.BlockSpec((HH, BE), lambda k: (0, k)),
        ],
        out_shape=[
            jax.ShapeDtypeStruct((E // 4, 128), jnp.float32),
            jax.ShapeDtypeStruct((HH, E), jnp.bfloat16),
        ],
        compiler_params=pltpu.CompilerParams(
            vmem_limit_bytes=56 * 1024 * 1024),
    )


def _make_gru(N, H, NB):
    def gru_body(h_ref, p_ref, w1_ref, w2_ref, b1_ref, out_ref):
        h = h_ref[...]
        m = p_ref[0] + p_ref[1]
        u = jnp.dot(h, w1_ref[...], preferred_element_type=jnp.float32)
        u = u + b1_ref[...]
        v = jnp.dot(m, w2_ref[...], preferred_element_type=jnp.float32)
        r = jax.nn.sigmoid(u[:, 0:H] + v[:, 0:H])
        z = jax.nn.sigmoid(u[:, H:2 * H] + v[:, H:2 * H])
        n = jnp.tanh(u[:, 2 * H:3 * H] + v[:, 2 * H:3 * H]
                     + r * u[:, 3 * H:4 * H])
        out_ref[...] = (1.0 - z) * n + z * h

    grid = (N // NB,)
    return pl.pallas_call(
        gru_body,
        grid=grid,
        in_specs=[
            pl.BlockSpec((NB, H), lambda k: (k, 0)),
            pl.BlockSpec((2, NB, H), lambda k: (0, k, 0)),
            pl.BlockSpec((H, 4 * H), lambda k: (0, 0)),
            pl.BlockSpec((H, 3 * H), lambda k: (0, 0)),
            pl.BlockSpec((1, 4 * H), lambda k: (0, 0)),
        ],
        out_specs=pl.BlockSpec((NB, H), lambda k: (k, 0)),
        out_shape=jax.ShapeDtypeStruct((N, H), jnp.float32),
    )


def kernel(x, Esrc, Etgt, edge_data, W_ih, W_hh, b_ih, b_hh):
    N, H = x.shape
    E = Esrc.shape[0]
    T = 8
    SB = 40

    edt_f32 = edge_data.reshape(E, H * H).T
    PB = 3200
    def _perm(a):
        return a.reshape(E // PB, 4, PB // 4).swapaxes(1, 2).reshape(E)
    esrc_p = _perm(Esrc)
    etgt2 = _perm(Etgt).reshape(E // SB, SB)
    zeros_n = jnp.zeros((N, H), jnp.float32)

    lane = lax.broadcasted_iota(jnp.int32, (H, H * H), 1)
    col = lax.broadcasted_iota(jnp.int32, (H, H * H), 0)
    r2 = (col == lane // H).astype(jnp.bfloat16)

    A = W_ih[:, :H].T
    B = W_ih[:, H:].T
    C = W_hh.T
    w1 = jnp.concatenate([A[:, :H] + C[:, :H],
                          A[:, H:2 * H] + C[:, H:2 * H],
                          A[:, 2 * H:],
                          C[:, 2 * H:]], axis=1)
    w2 = B
    b1 = jnp.concatenate([b_ih[:H] + b_hh[:H],
                          b_ih[H:2 * H] + b_hh[H:2 * H],
                          b_ih[2 * H:],
                          b_hh[2 * H:]])[None, :]

    gather_k = _make_gather(N, E, H)
    scatter_k = _make_scatter(N, E, H)
    bmm0_k = _make_bmm_first(E, H, BE=PB, PB=PB)
    bmm_k = _make_bmm(E, H, BE=2 * PB, PB=PB)
    gru_k = _make_gru(N, H, NB=2000)

    h = x
    edt_bf = None
    for t in range(T):
        sup = gather_k(h, esrc_p)
        if t == 0:
            msg4, edt_bf = bmm0_k(edt_f32, sup.reshape(E // 4, 128), r2)
        else:
            msg4 = bmm_k(edt_bf, edt_bf, sup.reshape(E // 4, 128), r2)
        parts = scatter_k(msg4.reshape(E, H), etgt2, zeros_n)
        h = gru_k(h, parts, w1, w2, b1)
    return h


# speedup vs baseline: 1.0898x; 1.0898x over previous
import functools

import jax
import jax.numpy as jnp
from jax import lax
from jax.experimental import pallas as pl
from jax.experimental.pallas import tpu as pltpu
from jax.experimental.pallas import tpu_sc as plsc

NC = 2
NS = 16
NW = NC * NS


def _make_gather(N, E, H):
    EPW = E // NW
    GCH = 1000
    NCH = EPW // GCH
    assert EPW % GCH == 0 and EPW % 8 == 0 and GCH % 8 == 0

    mesh = plsc.VectorSubcoreMesh(core_axis_name="c", subcore_axis_name="s")

    @functools.partial(
        pl.kernel, mesh=mesh,
        out_type=jax.ShapeDtypeStruct((E, H), jnp.float32),
        scratch_types=[
            pltpu.VMEM((EPW,), jnp.int32),
            pltpu.VMEM((2, GCH, H), jnp.float32),
            pltpu.SemaphoreType.DMA,
            pltpu.SemaphoreType.DMA,
            pltpu.SemaphoreType.DMA,
            pltpu.SemaphoreType.DMA,
        ],
        compiler_params=pltpu.CompilerParams(use_tc_tiling_on_sc=False),
    )
    def gather_k(h_hbm, esrc_hbm, out_hbm, idx_v, rows_v, g0, g1, w0, w1):
        c = lax.axis_index("c")
        s = lax.axis_index("s")
        w = c * NS + s
        base = w * EPW
        gsem = (g0, g1)
        wsem = (w0, w1)
        pltpu.sync_copy(esrc_hbm.at[pl.ds(base, EPW)], idx_v)
        gd = [None, None]
        wd = [None, None]
        prev = None
        for k in range(NCH):
            b = k & 1
            if k >= 2:
                wd[b].wait()
            gd[b] = pltpu.async_copy(
                h_hbm.at[idx_v.at[pl.ds(k * GCH, GCH)]], rows_v.at[b], gsem[b])
            if prev is not None:
                pk, pb = prev
                gd[pb].wait()
                wd[pb] = pltpu.async_copy(
                    rows_v.at[pb], out_hbm.at[pl.ds(base + pk * GCH, GCH)],
                    wsem[pb])
            prev = (k, b)
        pk, pb = prev
        gd[pb].wait()
        wd[pb] = pltpu.async_copy(
            rows_v.at[pb], out_hbm.at[pl.ds(base + pk * GCH, GCH)], wsem[pb])
        for b in range(2):
            if wd[b] is not None:
                wd[b].wait()

    return gather_k


def _make_scatter(N, E, H):
    SB = 40
    RPW = (E // SB) // NW
    MCH = 1000
    IRC = MCH // SB
    NCH = (RPW * SB) // MCH
    NPT = N // NS
    assert RPW * SB * NW == E and NCH * MCH == RPW * SB and N % NS == 0

    mesh = plsc.VectorSubcoreMesh(core_axis_name="c", subcore_axis_name="s")

    @functools.partial(
        pl.kernel, mesh=mesh,
        out_type=jax.ShapeDtypeStruct((NC, N, H), jnp.float32),
        scratch_types=[
            pltpu.VMEM((2, MCH, H), jnp.float32),
            pltpu.VMEM((2, IRC, SB), jnp.int32),
            pltpu.VMEM_SHARED((N, H), jnp.float32),
            pltpu.SemaphoreType.DMA,
            pltpu.SemaphoreType.DMA,
            pltpu.SemaphoreType.DMA,
            pltpu.SemaphoreType.DMA,
            pltpu.SemaphoreType.DMA,
        ],
        compiler_params=pltpu.CompilerParams(use_tc_tiling_on_sc=False),
    )
    def scatter_k(msg_hbm, etgt2_hbm, zeros_hbm, out_hbm,
                  mbuf, ibuf, acc_sh, m0, m1, i0, i1, ssem):
        c = lax.axis_index("c")
        s = lax.axis_index("s")
        w = c * NS + s
        ebase = w * RPW * SB
        rbase = w * RPW
        msem = (m0, m1)
        isem = (i0, i1)
        pltpu.sync_copy(zeros_hbm.at[pl.ds(s * NPT, NPT)],
                        acc_sh.at[pl.ds(s * NPT, NPT)])
        plsc.subcore_barrier()

        def start_stage(k, b):
            d1 = pltpu.async_copy(
                msg_hbm.at[pl.ds(ebase + k * MCH, MCH)], mbuf.at[b], msem[b])
            d2 = pltpu.async_copy(
                etgt2_hbm.at[pl.ds(rbase + k * IRC, IRC)], ibuf.at[b], isem[b])
            return (d1, d2)

        def fire_scatters(b):
            descs = []
            for j in range(IRC):
                descs.append(pltpu.async_copy(
                    mbuf.at[b].at[pl.ds(j * SB, SB)],
                    acc_sh.at[ibuf.at[b].at[j]],
                    ssem, add=True))
            return descs

        scat = {}
        prev = None
        for k in range(NCH):
            b = k & 1
            if k >= 2:
                for d in scat.pop(k - 2):
                    d.wait()
            sd = start_stage(k, b)
            if prev is not None:
                pk, pb, psd = prev
                psd[0].wait()
                psd[1].wait()
                scat[pk] = fire_scatters(pb)
            prev = (k, b, sd)
        pk, pb, psd = prev
        psd[0].wait()
        psd[1].wait()
        scat[pk] = fire_scatters(pb)
        for k in sorted(scat):
            for d in scat[k]:
                d.wait()
        plsc.subcore_barrier()
        pltpu.sync_copy(acc_sh.at[pl.ds(s * NPT, NPT)],
                        out_hbm.at[c].at[pl.ds(s * NPT, NPT)])

    return scatter_k


def _bmm_sub(ed_halves, sblk, r2, H, PB):
    Q = PB // 4
    sT = jnp.swapaxes(sblk, 0, 1)
    supt = jnp.concatenate(
        [sT[q * H:(q + 1) * H, :] for q in range(4)],
        axis=1).astype(jnp.bfloat16)
    mt = None
    c0 = 0
    for ed in ed_halves:
        rows = ed.shape[0] * H
        prod = (ed * supt[None]).reshape(rows, PB)
        part = jnp.dot(r2[:, c0:c0 + rows], prod,
                       preferred_element_type=jnp.float32)
        mt = part if mt is None else mt + part
        c0 += rows
    outs = [jnp.swapaxes(mt[:, q * Q:(q + 1) * Q], 0, 1)
            for q in range(4)]
    return jnp.concatenate(outs, axis=1)


def _make_bmm(E, H, BE, PB):
    HH = H * H
    NSUB = BE // PB

    def bmm_body(edt_ref, sup4_ref, r2_ref, out_ref):
        for t in range(NSUB):
            ed3 = edt_ref[:, t * PB:(t + 1) * PB].reshape(H, H, PB)
            sblk = sup4_ref[t * (PB // 4):(t + 1) * (PB // 4), :]
            out_ref[t * (PB // 4):(t + 1) * (PB // 4), :] = _bmm_sub(
                [ed3], sblk, r2_ref[...], H, PB)

    grid = (E // BE,)
    return pl.pallas_call(
        bmm_body,
        grid=grid,
        in_specs=[
            pl.BlockSpec((HH, BE), lambda k: (0, k)),
            pl.BlockSpec((BE // 4, 128), lambda k: (k, 0)),
            pl.BlockSpec((H, HH), lambda k: (0, 0)),
        ],
        out_specs=pl.BlockSpec((BE // 4, 128), lambda k: (k, 0)),
        out_shape=jax.ShapeDtypeStruct((E // 4, 128), jnp.float32),
        compiler_params=pltpu.CompilerParams(
            vmem_limit_bytes=56 * 1024 * 1024),
    )


def _make_bmm_first(E, H, BE, PB):
    HH = H * H
    NSUB = BE // PB

    def bmm_body(edt_ref, sup4_ref, r2_ref, out_ref, edtbf_ref):
        for t in range(NSUB):
            ed3 = edt_ref[:, t * PB:(t + 1) * PB].reshape(H, H, PB)
            ed_bf = ed3.astype(jnp.bfloat16)
            edtbf_ref[:, t * PB:(t + 1) * PB] = ed_bf.reshape(HH, PB)
            sblk = sup4_ref[t * (PB // 4):(t + 1) * (PB // 4), :]
            out_ref[t * (PB // 4):(t + 1) * (PB // 4), :] = _bmm_sub(
                [ed_bf], sblk, r2_ref[...], H, PB)

    grid = (E // BE,)
    return pl.pallas_call(
        bmm_body,
        grid=grid,
        in_specs=[
            pl.BlockSpec((HH, BE), lambda k: (0, k)),
            pl.BlockSpec((BE // 4, 128), lambda k: (k, 0)),
            pl.BlockSpec((H, HH), lambda k: (0, 0)),
        ],
        out_specs=[
            pl.BlockSpec((BE // 4, 128), lambda k: (k, 0)),
            pl.BlockSpec((HH, BE), lambda k: (0, k)),
        ],
        out_shape=[
            jax.ShapeDtypeStruct((E // 4, 128), jnp.float32),
            jax.ShapeDtypeStruct((HH, E), jnp.bfloat16),
        ],
        compiler_params=pltpu.CompilerParams(
            vmem_limit_bytes=56 * 1024 * 1024),
    )


def _make_gru(N, H, NB):
    NB4 = NB // 4

    def gru_body(h_ref, p_ref, w1t_ref, w2t_ref, b1t_ref, out_ref):
        hT = jnp.swapaxes(h_ref[...], 0, 1)
        mT = jnp.swapaxes(p_ref[0] + p_ref[1], 0, 1)
        b = b1t_ref[...]
        outs = []
        for q in range(4):
            hq = hT[q * H:(q + 1) * H, :]
            mq = mT[q * H:(q + 1) * H, :]
            u = jnp.dot(w1t_ref[...], hq,
                        preferred_element_type=jnp.float32) + b
            v = jnp.dot(w2t_ref[...], mq,
                        preferred_element_type=jnp.float32)
            r = jax.nn.sigmoid(u[0:H] + v[0:H])
            z = jax.nn.sigmoid(u[H:2 * H] + v[H:2 * H])
            n = jnp.tanh(u[2 * H:3 * H] + v[2 * H:3 * H]
                         + r * u[3 * H:4 * H])
            outs.append(jnp.swapaxes((1.0 - z) * n + z * hq, 0, 1))
        out_ref[...] = jnp.concatenate(outs, axis=1)

    grid = (N // NB,)
    return pl.pallas_call(
        gru_body,
        grid=grid,
        in_specs=[
            pl.BlockSpec((NB4, 128), lambda k: (k, 0)),
            pl.BlockSpec((2, NB4, 128), lambda k: (0, k, 0)),
            pl.BlockSpec((4 * H, H), lambda k: (0, 0)),
            pl.BlockSpec((3 * H, H), lambda k: (0, 0)),
            pl.BlockSpec((4 * H, NB4), lambda k: (0, 0)),
        ],
        out_specs=pl.BlockSpec((NB4, 128), lambda k: (k, 0)),
        out_shape=jax.ShapeDtypeStruct((N // 4, 128), jnp.float32),
    )


def kernel(x, Esrc, Etgt, edge_data, W_ih, W_hh, b_ih, b_hh):
    N, H = x.shape
    E = Esrc.shape[0]
    T = 8
    SB = 40

    edt_f32 = edge_data.reshape(E, H * H).T
    PB = 3200
    def _perm(a):
        return a.reshape(E // PB, 4, PB // 4).swapaxes(1, 2).reshape(E)
    esrc_p = _perm(Esrc)
    etgt2 = _perm(Etgt).reshape(E // SB, SB)
    zeros_n = jnp.zeros((N, H), jnp.float32)

    lane = lax.broadcasted_iota(jnp.int32, (H, H * H), 1)
    col = lax.broadcasted_iota(jnp.int32, (H, H * H), 0)
    r2 = (col == lane // H).astype(jnp.bfloat16)

    A = W_ih[:, :H].T
    B = W_ih[:, H:].T
    C = W_hh.T
    w1 = jnp.concatenate([A[:, :H] + C[:, :H],
                          A[:, H:2 * H] + C[:, H:2 * H],
                          A[:, 2 * H:],
                          C[:, 2 * H:]], axis=1)
    NB = N
    w1t = w1.T
    w2t = B.T
    b1v = jnp.concatenate([b_ih[:H] + b_hh[:H],
                           b_ih[H:2 * H] + b_hh[H:2 * H],
                           b_ih[2 * H:],
                           b_hh[2 * H:]])
    b1t = jnp.broadcast_to(b1v[:, None], (4 * H, NB // 4))

    gather_k = _make_gather(N, E, H)
    scatter_k = _make_scatter(N, E, H)
    bmm0_k = _make_bmm_first(E, H, BE=PB, PB=PB)
    bmm_k = _make_bmm(E, H, BE=2 * PB, PB=PB)
    gru_k = _make_gru(N, H, NB=NB)

    h4 = x.reshape(N // 4, 4 * H)
    edt_bf = None
    for t in range(T):
        sup = gather_k(h4.reshape(N, H), esrc_p)
        if t == 0:
            msg4, edt_bf = bmm0_k(edt_f32, sup.reshape(E // 4, 128), r2)
        else:
            msg4 = bmm_k(edt_bf, sup.reshape(E // 4, 128), r2)
        parts = scatter_k(msg4.reshape(E, H), etgt2, zeros_n)
        h4 = gru_k(h4, parts.reshape(2, N // 4, 4 * H), w1t, w2t, b1t)
    return h4.reshape(N, H)
